# Initial kernel scaffold; baseline (speedup 1.0000x reference)
import jax
import jax.numpy as jnp
from jax.experimental import pallas as pl


def kernel(x, A, W_l, W_r, att, bias):
    raise NotImplementedError("write your pallas kernel here")


# speedup vs baseline: 2.7855x; 2.7855x over previous
import jax
import jax.numpy as jnp
from jax.experimental import pallas as pl

N_NODES = 10000
D_IN = 128
D_HID = 16
NEG_SLOPE = 0.2


def _mm_body(x_ref, w_ref, o_ref):
    o_ref[...] = jnp.dot(x_ref[...], w_ref[...],
                         preferred_element_type=jnp.float32)


def _matmul(x, w):
    m, k = x.shape
    n = w.shape[1]
    blk = 2000
    return pl.pallas_call(
        _mm_body,
        grid=(m // blk,),
        in_specs=[pl.BlockSpec((blk, k), lambda i: (i, 0)),
                  pl.BlockSpec((k, n), lambda i: (0, 0))],
        out_specs=pl.BlockSpec((blk, n), lambda i: (i, 0)),
        out_shape=jax.ShapeDtypeStruct((m, n), jnp.float32),
    )(x, w)


def kernel(x, A, W_l, W_r, att, bias):
    src = A[0]
    dst = A[1]
    W = jnp.concatenate([W_l, W_r], axis=1)
    xlr = _matmul(x, W)
    x_l = xlr[:, :D_HID]
    x_r = xlr[:, D_HID:]
    h = x_l[src] + x_r[dst]
    h = jnp.where(h > 0, h, NEG_SLOPE * h)
    e = h @ att
    ex = jnp.exp(e)
    denom = jax.ops.segment_sum(ex, dst, num_segments=N_NODES)
    acc = jax.ops.segment_sum(ex[:, None] * x_l[src], dst, num_segments=N_NODES)
    return acc / (denom[:, None] + 1e-16) + bias


# speedup vs baseline: 21.3723x; 7.6728x over previous
import jax
import jax.numpy as jnp
from jax import lax
from jax.experimental import pallas as pl
from jax.experimental.pallas import tpu as pltpu
from jax.experimental.pallas import tpu_sc as plsc

N_NODES = 10000
N_EDGES = 320000
D_IN = 128
D_HID = 16
NEG_SLOPE = 0.2

NW = 32
CHUNK = 2048
NCHUNK = 5
NKCH = NW * NCHUNK
EPAD = NKCH * CHUNK
NPAD = N_NODES + 16
NGRP = CHUNK // 16
IDXW = 128
NIDX = CHUNK // IDXW
ROWS_PT = 624


def _mm_body(x_ref, wl_ref, wr_ref, ol_ref, or_ref):
    xv = x_ref[...]
    ol_ref[...] = jnp.dot(xv, wl_ref[...], preferred_element_type=jnp.float32)
    or_ref[...] = jnp.dot(xv, wr_ref[...], preferred_element_type=jnp.float32)


def _comb_body(acc_ref, den_ref, bias_ref, o_ref):
    a = acc_ref[0] + acc_ref[1]
    d = den_ref[0, :, 0] + den_ref[1, :, 0] + 1e-16
    o_ref[...] = a / d[:, None] + bias_ref[...]


def _edge_body(xl_hbm, xr_hbm, src_hbm, dst_hbm, att_hbm, zf_hbm,
               acc_out, den_out,
               sidx, didx, xl_buf, xr_buf, ebuf, attb,
               acc_sh, den_sh, sem1, sem2):
    c = lax.axis_index("c")
    s = lax.axis_index("s")
    wid = s * 2 + c

    pltpu.sync_copy(zf_hbm.at[pl.ds(s * ROWS_PT, ROWS_PT)],
                    xl_buf.at[pl.ds(0, ROWS_PT)])
    pltpu.sync_copy(xl_buf.at[pl.ds(0, ROWS_PT)],
                    acc_sh.at[pl.ds(s * ROWS_PT, ROWS_PT)])
    pltpu.sync_copy(xl_buf.at[pl.ds(0, ROWS_PT)],
                    den_sh.at[pl.ds(s * ROWS_PT, ROWS_PT)])

    @pl.when(s == 15)
    def _zero_tail():
        pltpu.sync_copy(zf_hbm.at[pl.ds(16 * ROWS_PT, 32)],
                        xl_buf.at[pl.ds(0, 32)])
        pltpu.sync_copy(xl_buf.at[pl.ds(0, 32)],
                        acc_sh.at[pl.ds(16 * ROWS_PT, 32)])
        pltpu.sync_copy(xl_buf.at[pl.ds(0, 32)],
                        den_sh.at[pl.ds(16 * ROWS_PT, 32)])

    pltpu.sync_copy(att_hbm, attb)
    attv = attb[...]
    iota = lax.iota(jnp.int32, 16)
    zer16 = jnp.zeros((16,), jnp.int32)
    zerf = jnp.zeros((16,), jnp.float32)

    def _zrow(r, carry):
        plsc.store_scatter(ebuf, [jnp.full((16,), r, jnp.int32), iota], zerf)
        return carry

    lax.fori_loop(0, CHUNK, _zrow, 0)
    plsc.subcore_barrier()

    for ch in range(NCHUNK):
        k = ch * NW + wid
        pltpu.sync_copy(src_hbm.at[k], sidx)
        pltpu.sync_copy(dst_hbm.at[k], didx)
        cps = []
        for j in range(NIDX):
            cps.append(pltpu.async_copy(
                xl_hbm.at[sidx.at[j]],
                xl_buf.at[pl.ds(j * IDXW, IDXW)], sem1))
            cps.append(pltpu.async_copy(
                xr_hbm.at[didx.at[j]],
                xr_buf.at[pl.ds(j * IDXW, IDXW)], sem2))
        for cp in cps:
            cp.wait()

        def grp(g, carry):
            ids = g * 16 + iota
            eacc = jnp.zeros((16,), jnp.float32)
            for i in range(16):
                fi = jnp.full((16,), i, jnp.int32)
                li = plsc.load_gather(xl_buf, [ids, fi])
                ri = plsc.load_gather(xr_buf, [ids, fi])
                h = li + ri
                h = jnp.where(h > 0, h, NEG_SLOPE * h)
                eacc = eacc + h * attv[i]
            exv = jnp.exp(eacc)
            plsc.store_scatter(ebuf, [ids, zer16], exv)
            for i in range(16):
                fi = jnp.full((16,), i, jnp.int32)
                li = plsc.load_gather(xl_buf, [ids, fi])
                plsc.store_scatter(xl_buf, [ids, fi], li * exv)
            return carry

        lax.fori_loop(0, NGRP, grp, 0)

        for j in range(NIDX):
            pltpu.sync_copy(xl_buf.at[pl.ds(j * IDXW, IDXW)],
                            acc_sh.at[didx.at[j]], add=True)
            pltpu.sync_copy(ebuf.at[pl.ds(j * IDXW, IDXW)],
                            den_sh.at[didx.at[j]], add=True)

    plsc.subcore_barrier()

    pltpu.sync_copy(acc_sh.at[pl.ds(s * ROWS_PT, ROWS_PT)],
                    xl_buf.at[pl.ds(0, ROWS_PT)])
    pltpu.sync_copy(xl_buf.at[pl.ds(0, ROWS_PT)],
                    acc_out.at[c].at[pl.ds(s * ROWS_PT, ROWS_PT)])
    pltpu.sync_copy(den_sh.at[pl.ds(s * ROWS_PT, ROWS_PT)],
                    xr_buf.at[pl.ds(0, ROWS_PT)])
    pltpu.sync_copy(xr_buf.at[pl.ds(0, ROWS_PT)],
                    den_out.at[c].at[pl.ds(s * ROWS_PT, ROWS_PT)])

    @pl.when(s == 15)
    def _out_tail():
        pltpu.sync_copy(acc_sh.at[pl.ds(16 * ROWS_PT, 16)],
                        xl_buf.at[pl.ds(0, 16)])
        pltpu.sync_copy(xl_buf.at[pl.ds(0, 16)],
                        acc_out.at[c].at[pl.ds(16 * ROWS_PT, 16)])
        pltpu.sync_copy(den_sh.at[pl.ds(16 * ROWS_PT, 16)],
                        xr_buf.at[pl.ds(0, 16)])
        pltpu.sync_copy(xr_buf.at[pl.ds(0, 16)],
                        den_out.at[c].at[pl.ds(16 * ROWS_PT, 16)])


_edge_kernel = pl.kernel(
    _edge_body,
    out_type=[jax.ShapeDtypeStruct((2, NPAD, D_HID), jnp.float32),
              jax.ShapeDtypeStruct((2, NPAD, D_HID), jnp.float32)],
    mesh=plsc.VectorSubcoreMesh(core_axis_name="c", subcore_axis_name="s"),
    compiler_params=pltpu.CompilerParams(use_tc_tiling_on_sc=False,
                                         needs_layout_passes=False),
    scratch_types=[
        pltpu.VMEM((NIDX, IDXW), jnp.int32),
        pltpu.VMEM((NIDX, IDXW), jnp.int32),
        pltpu.VMEM((CHUNK, D_HID), jnp.float32),
        pltpu.VMEM((CHUNK, D_HID), jnp.float32),
        pltpu.VMEM((CHUNK, D_HID), jnp.float32),
        pltpu.VMEM((16,), jnp.float32),
        pltpu.VMEM_SHARED((NPAD, D_HID), jnp.float32),
        pltpu.VMEM_SHARED((NPAD, D_HID), jnp.float32),
        pltpu.SemaphoreType.DMA,
        pltpu.SemaphoreType.DMA,
    ],
)


def kernel(x, A, W_l, W_r, att, bias):
    npad_e = EPAD - N_EDGES
    src = jnp.concatenate(
        [A[0].astype(jnp.int32), jnp.zeros((npad_e,), jnp.int32)])
    dst = jnp.concatenate(
        [A[1].astype(jnp.int32),
         jnp.full((npad_e,), N_NODES, jnp.int32)])
    src = src.reshape(NKCH, NIDX, IDXW)
    dst = dst.reshape(NKCH, NIDX, IDXW)
    blk = 2000
    x_l, x_r = pl.pallas_call(
        _mm_body,
        grid=(N_NODES // blk,),
        in_specs=[pl.BlockSpec((blk, D_IN), lambda i: (i, 0)),
                  pl.BlockSpec((D_IN, D_HID), lambda i: (0, 0)),
                  pl.BlockSpec((D_IN, D_HID), lambda i: (0, 0))],
        out_specs=[pl.BlockSpec((blk, D_HID), lambda i: (i, 0)),
                   pl.BlockSpec((blk, D_HID), lambda i: (i, 0))],
        out_shape=[jax.ShapeDtypeStruct((N_NODES, D_HID), jnp.float32),
                   jax.ShapeDtypeStruct((N_NODES, D_HID), jnp.float32)],
    )(x, W_l, W_r)
    x_l = jnp.pad(x_l, ((0, NPAD - N_NODES), (0, 0)))
    x_r = jnp.pad(x_r, ((0, NPAD - N_NODES), (0, 0)))
    zf = jnp.zeros((NPAD, D_HID), jnp.float32)
    acc_p, den_p = _edge_kernel(x_l, x_r, src, dst, att, zf)
    out = pl.pallas_call(
        _comb_body,
        grid=(1,),
        in_specs=[pl.BlockSpec((2, N_NODES, D_HID), lambda i: (0, 0, 0)),
                  pl.BlockSpec((2, N_NODES, D_HID), lambda i: (0, 0, 0)),
                  pl.BlockSpec((D_HID,), lambda i: (0,))],
        out_specs=pl.BlockSpec((N_NODES, D_HID), lambda i: (0, 0)),
        out_shape=jax.ShapeDtypeStruct((N_NODES, D_HID), jnp.float32),
    )(acc_p, den_p, bias)
    return out


# speedup vs baseline: 28.5945x; 1.3379x over previous
import jax
import jax.numpy as jnp
from jax import lax
from jax.experimental import pallas as pl
from jax.experimental.pallas import tpu as pltpu
from jax.experimental.pallas import tpu_sc as plsc

N_NODES = 10000
N_EDGES = 320000
D_IN = 128
D_HID = 16
NEG_SLOPE = 0.2

NW = 32
CHUNK = 2048
NCHUNK = 5
NKCH = NW * NCHUNK
EPAD = NKCH * CHUNK
NPAD = N_NODES + 16
NGRP = CHUNK // 16
IDXW = 128
NIDX = CHUNK // IDXW
ROWS_PT = 624


def _mm_body(x_ref, wl_ref, wr_ref, ol_ref, or_ref):
    xv = x_ref[...]
    ol_ref[...] = jnp.dot(xv, wl_ref[...], preferred_element_type=jnp.float32)
    or_ref[...] = jnp.dot(xv, wr_ref[...], preferred_element_type=jnp.float32)


def _comb_body(acc_ref, den_ref, bias_ref, o_ref):
    a = acc_ref[0] + acc_ref[1]
    d = den_ref[0, :, 0] + den_ref[1, :, 0] + 1e-16
    o_ref[...] = a / d[:, None] + bias_ref[...]


def _edge_body(xl_hbm, xr_hbm, src_hbm, dst_hbm, att_hbm, zf_hbm,
               acc_out, den_out,
               sidx, didx, xl_buf, xr_buf, ebuf, attb,
               acc_sh, den_sh, sem1, sem2):
    c = lax.axis_index("c")
    s = lax.axis_index("s")
    wid = s * 2 + c

    pltpu.sync_copy(zf_hbm.at[pl.ds(s * ROWS_PT, ROWS_PT)],
                    xl_buf.at[pl.ds(0, ROWS_PT)])
    pltpu.sync_copy(xl_buf.at[pl.ds(0, ROWS_PT)],
                    acc_sh.at[pl.ds(s * ROWS_PT, ROWS_PT)])
    pltpu.sync_copy(xl_buf.at[pl.ds(0, ROWS_PT)],
                    den_sh.at[pl.ds(s * ROWS_PT, ROWS_PT)])

    @pl.when(s == 15)
    def _zero_tail():
        pltpu.sync_copy(zf_hbm.at[pl.ds(16 * ROWS_PT, 32)],
                        xl_buf.at[pl.ds(0, 32)])
        pltpu.sync_copy(xl_buf.at[pl.ds(0, 32)],
                        acc_sh.at[pl.ds(16 * ROWS_PT, 32)])
        pltpu.sync_copy(xl_buf.at[pl.ds(0, 32)],
                        den_sh.at[pl.ds(16 * ROWS_PT, 32)])

    pltpu.sync_copy(att_hbm, attb)
    attv = attb[...]
    iota = lax.iota(jnp.int32, 16)
    zer16 = jnp.zeros((16,), jnp.int32)
    zerf = jnp.zeros((16,), jnp.float32)

    pltpu.sync_copy(zf_hbm.at[pl.ds(0, CHUNK)], ebuf)
    plsc.subcore_barrier()

    for ch in range(NCHUNK):
        k = ch * NW + wid
        pltpu.sync_copy(src_hbm.at[k], sidx)
        pltpu.sync_copy(dst_hbm.at[k], didx)
        cps = []
        for j in range(NIDX):
            cps.append(pltpu.async_copy(
                xl_hbm.at[sidx.at[j]],
                xl_buf.at[pl.ds(j * IDXW, IDXW)], sem1))
            cps.append(pltpu.async_copy(
                xr_hbm.at[didx.at[j]],
                xr_buf.at[pl.ds(j * IDXW, IDXW)], sem2))
        for cp in cps:
            cp.wait()

        def grp(g, carry):
            ids = g * 16 + iota
            eacc = jnp.zeros((16,), jnp.float32)
            cols = []
            for i in range(16):
                fi = jnp.full((16,), i, jnp.int32)
                li = plsc.load_gather(xl_buf, [ids, fi])
                ri = plsc.load_gather(xr_buf, [ids, fi])
                cols.append(li)
                h = li + ri
                h = jnp.where(h > 0, h, NEG_SLOPE * h)
                eacc = eacc + h * attv[i]
            exv = jnp.exp(eacc)
            plsc.store_scatter(ebuf, [ids, zer16], exv)
            for i in range(16):
                fi = jnp.full((16,), i, jnp.int32)
                plsc.store_scatter(xl_buf, [ids, fi], cols[i] * exv)
            return carry

        lax.fori_loop(0, NGRP, grp, 0)

        scps = []
        for j in range(NIDX):
            scps.append(pltpu.async_copy(
                xl_buf.at[pl.ds(j * IDXW, IDXW)],
                acc_sh.at[didx.at[j]], sem1, add=True))
            scps.append(pltpu.async_copy(
                ebuf.at[pl.ds(j * IDXW, IDXW)],
                den_sh.at[didx.at[j]], sem2, add=True))
        for cp in scps:
            cp.wait()

    plsc.subcore_barrier()

    pltpu.sync_copy(acc_sh.at[pl.ds(s * ROWS_PT, ROWS_PT)],
                    xl_buf.at[pl.ds(0, ROWS_PT)])
    pltpu.sync_copy(xl_buf.at[pl.ds(0, ROWS_PT)],
                    acc_out.at[c].at[pl.ds(s * ROWS_PT, ROWS_PT)])
    pltpu.sync_copy(den_sh.at[pl.ds(s * ROWS_PT, ROWS_PT)],
                    xr_buf.at[pl.ds(0, ROWS_PT)])
    pltpu.sync_copy(xr_buf.at[pl.ds(0, ROWS_PT)],
                    den_out.at[c].at[pl.ds(s * ROWS_PT, ROWS_PT)])

    @pl.when(s == 15)
    def _out_tail():
        pltpu.sync_copy(acc_sh.at[pl.ds(16 * ROWS_PT, 16)],
                        xl_buf.at[pl.ds(0, 16)])
        pltpu.sync_copy(xl_buf.at[pl.ds(0, 16)],
                        acc_out.at[c].at[pl.ds(16 * ROWS_PT, 16)])
        pltpu.sync_copy(den_sh.at[pl.ds(16 * ROWS_PT, 16)],
                        xr_buf.at[pl.ds(0, 16)])
        pltpu.sync_copy(xr_buf.at[pl.ds(0, 16)],
                        den_out.at[c].at[pl.ds(16 * ROWS_PT, 16)])


_edge_kernel = pl.kernel(
    _edge_body,
    out_type=[jax.ShapeDtypeStruct((2, NPAD, D_HID), jnp.float32),
              jax.ShapeDtypeStruct((2, NPAD, D_HID), jnp.float32)],
    mesh=plsc.VectorSubcoreMesh(core_axis_name="c", subcore_axis_name="s"),
    compiler_params=pltpu.CompilerParams(use_tc_tiling_on_sc=False,
                                         needs_layout_passes=False),
    scratch_types=[
        pltpu.VMEM((NIDX, IDXW), jnp.int32),
        pltpu.VMEM((NIDX, IDXW), jnp.int32),
        pltpu.VMEM((CHUNK, D_HID), jnp.float32),
        pltpu.VMEM((CHUNK, D_HID), jnp.float32),
        pltpu.VMEM((CHUNK, D_HID), jnp.float32),
        pltpu.VMEM((16,), jnp.float32),
        pltpu.VMEM_SHARED((NPAD, D_HID), jnp.float32),
        pltpu.VMEM_SHARED((NPAD, D_HID), jnp.float32),
        pltpu.SemaphoreType.DMA,
        pltpu.SemaphoreType.DMA,
    ],
)


def kernel(x, A, W_l, W_r, att, bias):
    npad_e = EPAD - N_EDGES
    src = jnp.concatenate(
        [A[0].astype(jnp.int32), jnp.zeros((npad_e,), jnp.int32)])
    dst = jnp.concatenate(
        [A[1].astype(jnp.int32),
         jnp.full((npad_e,), N_NODES, jnp.int32)])
    src = src.reshape(NKCH, NIDX, IDXW)
    dst = dst.reshape(NKCH, NIDX, IDXW)
    blk = 2000
    x_l, x_r = pl.pallas_call(
        _mm_body,
        grid=(N_NODES // blk,),
        in_specs=[pl.BlockSpec((blk, D_IN), lambda i: (i, 0)),
                  pl.BlockSpec((D_IN, D_HID), lambda i: (0, 0)),
                  pl.BlockSpec((D_IN, D_HID), lambda i: (0, 0))],
        out_specs=[pl.BlockSpec((blk, D_HID), lambda i: (i, 0)),
                   pl.BlockSpec((blk, D_HID), lambda i: (i, 0))],
        out_shape=[jax.ShapeDtypeStruct((N_NODES, D_HID), jnp.float32),
                   jax.ShapeDtypeStruct((N_NODES, D_HID), jnp.float32)],
    )(x, W_l, W_r)
    x_l = jnp.pad(x_l, ((0, NPAD - N_NODES), (0, 0)))
    x_r = jnp.pad(x_r, ((0, NPAD - N_NODES), (0, 0)))
    zf = jnp.zeros((NPAD, D_HID), jnp.float32)
    acc_p, den_p = _edge_kernel(x_l, x_r, src, dst, att, zf)
    out = pl.pallas_call(
        _comb_body,
        grid=(1,),
        in_specs=[pl.BlockSpec((2, N_NODES, D_HID), lambda i: (0, 0, 0)),
                  pl.BlockSpec((2, N_NODES, D_HID), lambda i: (0, 0, 0)),
                  pl.BlockSpec((D_HID,), lambda i: (0,))],
        out_specs=pl.BlockSpec((N_NODES, D_HID), lambda i: (0, 0)),
        out_shape=jax.ShapeDtypeStruct((N_NODES, D_HID), jnp.float32),
    )(acc_p, den_p, bias)
    return out


# speedup vs baseline: 31.1160x; 1.0882x over previous
import jax
import jax.numpy as jnp
from jax import lax
from jax.experimental import pallas as pl
from jax.experimental.pallas import tpu as pltpu
from jax.experimental.pallas import tpu_sc as plsc

N_NODES = 10000
N_EDGES = 320000
D_IN = 128
D_HID = 16
NEG_SLOPE = 0.2

NW = 32
CHUNK = 2048
NCHUNK = 5
NKCH = NW * NCHUNK
EPAD = NKCH * CHUNK
NPAD = N_NODES + 16
NGRP = CHUNK // 16
IDXW = 128
NIDX = CHUNK // IDXW
ROWS_PT = 624
DROWS = 640
DROWS_PT = 40


def _mm_body(x_ref, wl_ref, wr_ref, ol_ref, or_ref):
    xv = x_ref[...]
    ol_ref[...] = jnp.dot(xv, wl_ref[...], preferred_element_type=jnp.float32)
    or_ref[...] = jnp.dot(xv, wr_ref[...], preferred_element_type=jnp.float32)


def _comb_body(acc_ref, den_ref, bias_ref, o_ref):
    a = acc_ref[0] + acc_ref[1]
    d = den_ref[0] + den_ref[1] + 1e-16
    o_ref[...] = a / d[:, None] + bias_ref[...]


def _edge_body(xl_hbm, xr_hbm, src_hbm, dst_hbm, att_hbm, zf_hbm, id_hbm,
               acc_out, den_out,
               sidx, didx, xl_buf, xr_buf, den_loc, identb, attb,
               acc_sh, den_sh, sem1, sem2):
    c = lax.axis_index("c")
    s = lax.axis_index("s")
    wid = s * 2 + c

    pltpu.sync_copy(zf_hbm.at[pl.ds(s * ROWS_PT, ROWS_PT)],
                    xl_buf.at[pl.ds(0, ROWS_PT)])
    pltpu.sync_copy(xl_buf.at[pl.ds(0, ROWS_PT)],
                    acc_sh.at[pl.ds(s * ROWS_PT, ROWS_PT)])
    pltpu.sync_copy(zf_hbm.at[pl.ds(s * DROWS_PT, DROWS_PT)],
                    xr_buf.at[pl.ds(0, DROWS_PT)])
    pltpu.sync_copy(xr_buf.at[pl.ds(0, DROWS_PT)],
                    den_sh.at[pl.ds(s * DROWS_PT, DROWS_PT)])

    @pl.when(s == 15)
    def _zero_tail():
        pltpu.sync_copy(zf_hbm.at[pl.ds(16 * ROWS_PT, 32)],
                        xl_buf.at[pl.ds(0, 32)])
        pltpu.sync_copy(xl_buf.at[pl.ds(0, 32)],
                        acc_sh.at[pl.ds(16 * ROWS_PT, 32)])

    pltpu.sync_copy(zf_hbm.at[pl.ds(0, DROWS)], den_loc)
    pltpu.sync_copy(id_hbm, identb)
    pltpu.sync_copy(att_hbm, attb)
    attv = attb[...]
    iota = lax.iota(jnp.int32, 16)
    plsc.subcore_barrier()

    for ch in range(NCHUNK):
        k = ch * NW + wid
        pltpu.sync_copy(src_hbm.at[k], sidx)
        pltpu.sync_copy(dst_hbm.at[k], didx)
        cps = []
        for j in range(NIDX):
            cps.append(pltpu.async_copy(
                xl_hbm.at[sidx.at[j]],
                xl_buf.at[pl.ds(j * IDXW, IDXW)], sem1))
            cps.append(pltpu.async_copy(
                xr_hbm.at[didx.at[j]],
                xr_buf.at[pl.ds(j * IDXW, IDXW)], sem2))
        for cp in cps:
            cp.wait()

        def grp(g, carry):
            ids = g * 16 + iota
            eacc = jnp.zeros((16,), jnp.float32)
            cols = []
            for i in range(16):
                fi = jnp.full((16,), i, jnp.int32)
                li = plsc.load_gather(xl_buf, [ids, fi])
                ri = plsc.load_gather(xr_buf, [ids, fi])
                cols.append(li)
                h = li + ri
                h = jnp.where(h > 0, h, NEG_SLOPE * h)
                eacc = eacc + h * attv[i]
            exv = jnp.exp(eacc)
            dvec = plsc.load_gather(didx, [ids >> 7, ids & 127])
            plsc.addupdate_scatter(den_loc, [dvec >> 4, dvec & 15], exv)
            for i in range(16):
                fi = jnp.full((16,), i, jnp.int32)
                plsc.store_scatter(xl_buf, [ids, fi], cols[i] * exv)
            return carry

        lax.fori_loop(0, NGRP, grp, 0)

        scps = []
        for j in range(NIDX):
            scps.append(pltpu.async_copy(
                xl_buf.at[pl.ds(j * IDXW, IDXW)],
                acc_sh.at[didx.at[j]], sem1, add=True))
        for cp in scps:
            cp.wait()

    scps = []
    for j in range(DROWS // IDXW):
        scps.append(pltpu.async_copy(
            den_loc.at[pl.ds(j * IDXW, IDXW)],
            den_sh.at[identb.at[j]], sem2, add=True))
    for cp in scps:
        cp.wait()

    plsc.subcore_barrier()

    pltpu.sync_copy(acc_sh.at[pl.ds(s * ROWS_PT, ROWS_PT)],
                    xl_buf.at[pl.ds(0, ROWS_PT)])
    pltpu.sync_copy(xl_buf.at[pl.ds(0, ROWS_PT)],
                    acc_out.at[c].at[pl.ds(s * ROWS_PT, ROWS_PT)])
    pltpu.sync_copy(den_sh.at[pl.ds(s * DROWS_PT, DROWS_PT)],
                    xr_buf.at[pl.ds(0, DROWS_PT)])
    pltpu.sync_copy(xr_buf.at[pl.ds(0, DROWS_PT)],
                    den_out.at[c].at[pl.ds(s * DROWS_PT, DROWS_PT)])

    @pl.when(s == 15)
    def _out_tail():
        pltpu.sync_copy(acc_sh.at[pl.ds(16 * ROWS_PT, 16)],
                        xl_buf.at[pl.ds(0, 16)])
        pltpu.sync_copy(xl_buf.at[pl.ds(0, 16)],
                        acc_out.at[c].at[pl.ds(16 * ROWS_PT, 16)])


_edge_kernel = pl.kernel(
    _edge_body,
    out_type=[jax.ShapeDtypeStruct((2, NPAD, D_HID), jnp.float32),
              jax.ShapeDtypeStruct((2, DROWS, D_HID), jnp.float32)],
    mesh=plsc.VectorSubcoreMesh(core_axis_name="c", subcore_axis_name="s"),
    compiler_params=pltpu.CompilerParams(use_tc_tiling_on_sc=False,
                                         needs_layout_passes=False),
    scratch_types=[
        pltpu.VMEM((NIDX, IDXW), jnp.int32),
        pltpu.VMEM((NIDX, IDXW), jnp.int32),
        pltpu.VMEM((CHUNK, D_HID), jnp.float32),
        pltpu.VMEM((CHUNK, D_HID), jnp.float32),
        pltpu.VMEM((DROWS, D_HID), jnp.float32),
        pltpu.VMEM((DROWS // IDXW, IDXW), jnp.int32),
        pltpu.VMEM((16,), jnp.float32),
        pltpu.VMEM_SHARED((NPAD, D_HID), jnp.float32),
        pltpu.VMEM_SHARED((DROWS, D_HID), jnp.float32),
        pltpu.SemaphoreType.DMA,
        pltpu.SemaphoreType.DMA,
    ],
)


def kernel(x, A, W_l, W_r, att, bias):
    npad_e = EPAD - N_EDGES
    src = jnp.concatenate(
        [A[0].astype(jnp.int32), jnp.zeros((npad_e,), jnp.int32)])
    dst = jnp.concatenate(
        [A[1].astype(jnp.int32),
         jnp.full((npad_e,), N_NODES, jnp.int32)])
    src = src.reshape(NKCH, NIDX, IDXW)
    dst = dst.reshape(NKCH, NIDX, IDXW)
    blk = 2000
    x_l, x_r = pl.pallas_call(
        _mm_body,
        grid=(N_NODES // blk,),
        in_specs=[pl.BlockSpec((blk, D_IN), lambda i: (i, 0)),
                  pl.BlockSpec((D_IN, D_HID), lambda i: (0, 0)),
                  pl.BlockSpec((D_IN, D_HID), lambda i: (0, 0))],
        out_specs=[pl.BlockSpec((blk, D_HID), lambda i: (i, 0)),
                   pl.BlockSpec((blk, D_HID), lambda i: (i, 0))],
        out_shape=[jax.ShapeDtypeStruct((N_NODES, D_HID), jnp.float32),
                   jax.ShapeDtypeStruct((N_NODES, D_HID), jnp.float32)],
    )(x, W_l, W_r)
    x_l = jnp.pad(x_l, ((0, NPAD - N_NODES), (0, 0)))
    x_r = jnp.pad(x_r, ((0, NPAD - N_NODES), (0, 0)))
    zf = jnp.zeros((NPAD, D_HID), jnp.float32)
    ident = jnp.arange(DROWS, dtype=jnp.int32).reshape(DROWS // IDXW, IDXW)
    acc_p, den_p = _edge_kernel(x_l, x_r, src, dst, att, zf, ident)
    den2 = den_p.reshape(2, DROWS * D_HID)[:, :N_NODES]
    out = pl.pallas_call(
        _comb_body,
        grid=(1,),
        in_specs=[pl.BlockSpec((2, N_NODES, D_HID), lambda i: (0, 0, 0)),
                  pl.BlockSpec((2, N_NODES), lambda i: (0, 0)),
                  pl.BlockSpec((D_HID,), lambda i: (0,))],
        out_specs=pl.BlockSpec((N_NODES, D_HID), lambda i: (0, 0)),
        out_shape=jax.ShapeDtypeStruct((N_NODES, D_HID), jnp.float32),
    )(acc_p, den2, bias)
    return out


# speedup vs baseline: 32.7231x; 1.0516x over previous
import jax
import jax.numpy as jnp
from jax import lax
from jax.experimental import pallas as pl
from jax.experimental.pallas import tpu as pltpu
from jax.experimental.pallas import tpu_sc as plsc

N_NODES = 10000
N_EDGES = 320000
D_IN = 128
D_HID = 16
NEG_SLOPE = 0.2

NW = 32
CHUNK = 2048
NCHUNK = 5
NKCH = NW * NCHUNK
EPAD = NKCH * CHUNK
NPAD = N_NODES + 16
NGRP = CHUNK // 16
IDXW = 128
NIDX = CHUNK // IDXW
ROWS_PT = 624
DROWS = 640
DROWS_PT = 40


def _mm_body(x_ref, wl_ref, wr_ref, ol_ref, or_ref):
    xv = x_ref[...]
    ol_ref[...] = jnp.dot(xv, wl_ref[...], preferred_element_type=jnp.float32)
    or_ref[...] = jnp.dot(xv, wr_ref[...], preferred_element_type=jnp.float32)


def _comb_body(acc_ref, den_ref, bias_ref, o_ref):
    a = acc_ref[0] + acc_ref[1]
    d = den_ref[0] + den_ref[1] + 1e-16
    o_ref[...] = a / d[:, None] + bias_ref[...]


def _edge_body(xl_hbm, xr_hbm, src_hbm, dst_hbm, att_hbm, zf_hbm, id_hbm,
               acc_out, den_out,
               sidx, didx, xl_buf, xr_buf, den_loc, identb, attb,
               acc_sh, den_sh, sem1, sem2):
    c = lax.axis_index("c")
    s = lax.axis_index("s")
    wid = s * 2 + c

    pltpu.sync_copy(zf_hbm.at[pl.ds(s * ROWS_PT, ROWS_PT)],
                    xl_buf.at[pl.ds(0, ROWS_PT)])
    pltpu.sync_copy(xl_buf.at[pl.ds(0, ROWS_PT)],
                    acc_sh.at[pl.ds(s * ROWS_PT, ROWS_PT)])
    pltpu.sync_copy(zf_hbm.at[pl.ds(s * DROWS_PT, DROWS_PT)],
                    xr_buf.at[pl.ds(0, DROWS_PT)])
    pltpu.sync_copy(xr_buf.at[pl.ds(0, DROWS_PT)],
                    den_sh.at[pl.ds(s * DROWS_PT, DROWS_PT)])

    @pl.when(s == 15)
    def _zero_tail():
        pltpu.sync_copy(zf_hbm.at[pl.ds(16 * ROWS_PT, 32)],
                        xl_buf.at[pl.ds(0, 32)])
        pltpu.sync_copy(xl_buf.at[pl.ds(0, 32)],
                        acc_sh.at[pl.ds(16 * ROWS_PT, 32)])

    pltpu.sync_copy(zf_hbm.at[pl.ds(0, DROWS)], den_loc)
    pltpu.sync_copy(id_hbm, identb)
    pltpu.sync_copy(att_hbm, attb)
    attv = attb[...]
    iota = lax.iota(jnp.int32, 16)
    plsc.subcore_barrier()

    for ch in range(NCHUNK):
        k = ch * NW + wid
        pltpu.sync_copy(src_hbm.at[k], sidx)
        pltpu.sync_copy(dst_hbm.at[k], didx)
        cps = []
        for j in range(NIDX):
            cps.append(pltpu.async_copy(
                xl_hbm.at[sidx.at[j]],
                xl_buf.at[pl.ds(j * IDXW, IDXW)], sem1))
            cps.append(pltpu.async_copy(
                xr_hbm.at[didx.at[j]],
                xr_buf.at[pl.ds(j * IDXW, IDXW)], sem2))
        for cp in cps:
            cp.wait()

        def grp(g, carry):
            ids = g * 16 + iota
            cols = []
            terms = []
            for i in range(16):
                fi = jnp.full((16,), i, jnp.int32)
                li = plsc.load_gather(xl_buf, [ids, fi])
                ri = plsc.load_gather(xr_buf, [ids, fi])
                cols.append(li)
                h = li + ri
                h = jnp.where(h > 0, h, NEG_SLOPE * h)
                terms.append(h * attv[i])
            while len(terms) > 1:
                terms = [a + b for a, b in zip(terms[::2], terms[1::2])]
            exv = jnp.exp(terms[0])
            dvec = plsc.load_gather(didx, [ids >> 7, ids & 127])
            plsc.addupdate_scatter(den_loc, [dvec >> 4, dvec & 15], exv)
            for i in range(16):
                fi = jnp.full((16,), i, jnp.int32)
                plsc.store_scatter(xl_buf, [ids, fi], cols[i] * exv)
            return carry

        lax.fori_loop(0, NGRP, grp, 0)

        scps = []
        for j in range(NIDX):
            scps.append(pltpu.async_copy(
                xl_buf.at[pl.ds(j * IDXW, IDXW)],
                acc_sh.at[didx.at[j]], sem1, add=True))
        for cp in scps:
            cp.wait()

    scps = []
    for j in range(DROWS // IDXW):
        scps.append(pltpu.async_copy(
            den_loc.at[pl.ds(j * IDXW, IDXW)],
            den_sh.at[identb.at[j]], sem2, add=True))
    for cp in scps:
        cp.wait()

    plsc.subcore_barrier()

    pltpu.sync_copy(acc_sh.at[pl.ds(s * ROWS_PT, ROWS_PT)],
                    xl_buf.at[pl.ds(0, ROWS_PT)])
    pltpu.sync_copy(xl_buf.at[pl.ds(0, ROWS_PT)],
                    acc_out.at[c].at[pl.ds(s * ROWS_PT, ROWS_PT)])
    pltpu.sync_copy(den_sh.at[pl.ds(s * DROWS_PT, DROWS_PT)],
                    xr_buf.at[pl.ds(0, DROWS_PT)])
    pltpu.sync_copy(xr_buf.at[pl.ds(0, DROWS_PT)],
                    den_out.at[c].at[pl.ds(s * DROWS_PT, DROWS_PT)])

    @pl.when(s == 15)
    def _out_tail():
        pltpu.sync_copy(acc_sh.at[pl.ds(16 * ROWS_PT, 16)],
                        xl_buf.at[pl.ds(0, 16)])
        pltpu.sync_copy(xl_buf.at[pl.ds(0, 16)],
                        acc_out.at[c].at[pl.ds(16 * ROWS_PT, 16)])


_edge_kernel = pl.kernel(
    _edge_body,
    out_type=[jax.ShapeDtypeStruct((2, NPAD, D_HID), jnp.float32),
              jax.ShapeDtypeStruct((2, DROWS, D_HID), jnp.float32)],
    mesh=plsc.VectorSubcoreMesh(core_axis_name="c", subcore_axis_name="s"),
    compiler_params=pltpu.CompilerParams(use_tc_tiling_on_sc=False,
                                         needs_layout_passes=False),
    scratch_types=[
        pltpu.VMEM((NIDX, IDXW), jnp.int32),
        pltpu.VMEM((NIDX, IDXW), jnp.int32),
        pltpu.VMEM((CHUNK, D_HID), jnp.float32),
        pltpu.VMEM((CHUNK, D_HID), jnp.float32),
        pltpu.VMEM((DROWS, D_HID), jnp.float32),
        pltpu.VMEM((DROWS // IDXW, IDXW), jnp.int32),
        pltpu.VMEM((16,), jnp.float32),
        pltpu.VMEM_SHARED((NPAD, D_HID), jnp.float32),
        pltpu.VMEM_SHARED((DROWS, D_HID), jnp.float32),
        pltpu.SemaphoreType.DMA,
        pltpu.SemaphoreType.DMA,
    ],
)


def kernel(x, A, W_l, W_r, att, bias):
    npad_e = EPAD - N_EDGES
    src = jnp.concatenate(
        [A[0].astype(jnp.int32), jnp.zeros((npad_e,), jnp.int32)])
    dst = jnp.concatenate(
        [A[1].astype(jnp.int32),
         jnp.full((npad_e,), N_NODES, jnp.int32)])
    src = src.reshape(NKCH, NIDX, IDXW)
    dst = dst.reshape(NKCH, NIDX, IDXW)
    blk = 2000
    x_l, x_r = pl.pallas_call(
        _mm_body,
        grid=(N_NODES // blk,),
        in_specs=[pl.BlockSpec((blk, D_IN), lambda i: (i, 0)),
                  pl.BlockSpec((D_IN, D_HID), lambda i: (0, 0)),
                  pl.BlockSpec((D_IN, D_HID), lambda i: (0, 0))],
        out_specs=[pl.BlockSpec((blk, D_HID), lambda i: (i, 0)),
                   pl.BlockSpec((blk, D_HID), lambda i: (i, 0))],
        out_shape=[jax.ShapeDtypeStruct((N_NODES, D_HID), jnp.float32),
                   jax.ShapeDtypeStruct((N_NODES, D_HID), jnp.float32)],
    )(x, W_l, W_r)
    x_l = jnp.pad(x_l, ((0, NPAD - N_NODES), (0, 0)))
    x_r = jnp.pad(x_r, ((0, NPAD - N_NODES), (0, 0)))
    zf = jnp.zeros((NPAD, D_HID), jnp.float32)
    ident = jnp.arange(DROWS, dtype=jnp.int32).reshape(DROWS // IDXW, IDXW)
    acc_p, den_p = _edge_kernel(x_l, x_r, src, dst, att, zf, ident)
    den2 = den_p.reshape(2, DROWS * D_HID)[:, :N_NODES]
    out = pl.pallas_call(
        _comb_body,
        grid=(1,),
        in_specs=[pl.BlockSpec((2, N_NODES, D_HID), lambda i: (0, 0, 0)),
                  pl.BlockSpec((2, N_NODES), lambda i: (0, 0)),
                  pl.BlockSpec((D_HID,), lambda i: (0,))],
        out_specs=pl.BlockSpec((N_NODES, D_HID), lambda i: (0, 0)),
        out_shape=jax.ShapeDtypeStruct((N_NODES, D_HID), jnp.float32),
    )(acc_p, den2, bias)
    return out
